# baseline (speedup 1.0000x reference)
import functools

import jax
import jax.numpy as jnp
from jax import lax
from jax.experimental import pallas as pl
from jax.experimental.pallas import tpu as pltpu
from jax.experimental.pallas import tpu_sc as plsc

VOCAB = 100000
D = 128
B = 16384
NSLOT = 5
IN_SIZE = NSLOT * D

NC = 2
NS = 16
NW = NC * NS
BPW = B // NW
SPC = 16
RPC = SPC * NSLOT
NCHUNK = BPW // SPC
NQ = D // 16


def _sc_kernel(g_feats, samples_flat, w_flat, bias16):
    mesh = plsc.VectorSubcoreMesh(core_axis_name="c", subcore_axis_name="s")

    @functools.partial(
        pl.kernel,
        mesh=mesh,
        out_type=jax.ShapeDtypeStruct((B,), jnp.float32),
        scratch_types=[
            pltpu.VMEM((BPW * NSLOT,), jnp.int32),
            pltpu.VMEM((4, RPC, D), jnp.float32),
            pltpu.VMEM((IN_SIZE,), jnp.float32),
            pltpu.VMEM((16,), jnp.float32),
            pltpu.VMEM((BPW,), jnp.float32),
            pltpu.SemaphoreType.DMA,
            pltpu.SemaphoreType.DMA,
            pltpu.SemaphoreType.DMA,
            pltpu.SemaphoreType.DMA,
        ],
    )
    def sc_k(g_hbm, sflat_hbm, w_hbm, bias_hbm, out_hbm,
             sv, dbuf, wv, bv, acc, sem0, sem1, sem2, sem3):
        wid = lax.axis_index("s") * NC + lax.axis_index("c")
        base = wid * BPW
        pltpu.sync_copy(sflat_hbm.at[pl.ds(base * NSLOT, BPW * NSLOT)], sv)
        pltpu.sync_copy(w_hbm, wv)
        pltpu.sync_copy(bias_hbm, bv)
        sems = (sem0, sem1, sem2, sem3)
        wreg = [[wv[pl.ds(j * D + q * 16, 16)] for q in range(NQ)]
                for j in range(NSLOT)]
        io = lax.iota(jnp.int32, 16)
        bias_v = bv[...]

        def fire(t, b):
            return pltpu.async_copy(
                g_hbm.at[sv.at[pl.ds(t * RPC, RPC)]], dbuf.at[b], sems[b]
            )

        def wait(b):
            pltpu.make_async_copy(
                g_hbm.at[pl.ds(0, RPC), :], dbuf.at[b], sems[b]
            ).wait()

        def tree_sum(ts):
            while len(ts) > 1:
                nxt = [ts[k] + ts[k + 1] for k in range(0, len(ts) - 1, 2)]
                if len(ts) % 2:
                    nxt.append(ts[-1])
                ts = nxt
            return ts[0]

        def compute(t, b):
            accs = [None] * SPC
            for j in range(NSLOT):
                for i in range(SPC):
                    t8 = tree_sum([
                        dbuf[b, NSLOT * i + j, pl.ds(q * 16, 16)] * wreg[j][q]
                        for q in range(NQ)
                    ])
                    accs[i] = t8 if accs[i] is None else accs[i] + t8
            res = bias_v
            for i in range(SPC):
                s = accs[i]
                for sh in (1, 2, 4, 8):
                    s = s + s[jnp.bitwise_xor(io, sh)]
                res = jnp.where(io == i, res + s, res)
            acc[pl.ds(t * 16, 16)] = res

        for b in range(4):
            fire(b, b)

        def body(it, carry):
            for b in range(4):
                t = it * 4 + b
                wait(b)
                compute(t, b)

                @pl.when(t + 4 < NCHUNK)
                def _():
                    fire(t + 4, b)

            return carry

        lax.fori_loop(0, NCHUNK // 4, body, 0)
        pltpu.sync_copy(acc, out_hbm.at[pl.ds(base, BPW)])

    return sc_k(g_feats, samples_flat, w_flat, bias16)


def kernel(g_feats, samples, W, b):
    samples_flat = samples.reshape(-1)
    w_flat = W.reshape(-1)
    bias16 = jnp.full((16,), b[0], jnp.float32)
    out_flat = _sc_kernel(g_feats, samples_flat, w_flat, bias16)
    return out_flat.reshape(B, 1)


# speedup vs baseline: 1.0958x; 1.0958x over previous
import functools

import jax
import jax.numpy as jnp
from jax import lax
from jax.experimental import pallas as pl
from jax.experimental.pallas import tpu as pltpu
from jax.experimental.pallas import tpu_sc as plsc

VOCAB = 100000
D = 128
B = 16384
NSLOT = 5
IN_SIZE = NSLOT * D

NC = 2
NS = 16
NW = NC * NS
BPW = B // NW
SPC = 16
RPC = SPC * NSLOT
NCHUNK = BPW // SPC
NQ = D // 16


def _sc_kernel(g_feats, samples_flat, w_flat, bias16):
    mesh = plsc.VectorSubcoreMesh(core_axis_name="c", subcore_axis_name="s")

    @functools.partial(
        pl.kernel,
        mesh=mesh,
        out_type=jax.ShapeDtypeStruct((B,), jnp.float32),
        scratch_types=[
            pltpu.VMEM((BPW * NSLOT,), jnp.int32),
            pltpu.VMEM((2, RPC, D), jnp.float32),
            pltpu.VMEM((IN_SIZE,), jnp.float32),
            pltpu.VMEM((16,), jnp.float32),
            pltpu.VMEM((BPW,), jnp.float32),
            pltpu.SemaphoreType.DMA,
            pltpu.SemaphoreType.DMA,
        ],
    )
    def sc_k(g_hbm, sflat_hbm, w_hbm, bias_hbm, out_hbm,
             sv, dbuf, wv, bv, acc, sem0, sem1):
        wid = lax.axis_index("s") * NC + lax.axis_index("c")
        base = wid * BPW
        pltpu.sync_copy(sflat_hbm.at[pl.ds(base * NSLOT, BPW * NSLOT)], sv)
        pltpu.sync_copy(w_hbm, wv)
        pltpu.sync_copy(bias_hbm, bv)
        sems = (sem0, sem1)
        io = lax.iota(jnp.int32, 16)
        bias_v = bv[...]

        def fire(t, b):
            return pltpu.async_copy(
                g_hbm.at[sv.at[pl.ds(t * RPC, RPC)]], dbuf.at[b], sems[b]
            )

        def wait(b):
            pltpu.make_async_copy(
                g_hbm.at[pl.ds(0, RPC), :], dbuf.at[b], sems[b]
            ).wait()

        def tree_sum(ts):
            while len(ts) > 1:
                nxt = [ts[k] + ts[k + 1] for k in range(0, len(ts) - 1, 2)]
                if len(ts) % 2:
                    nxt.append(ts[-1])
                ts = nxt
            return ts[0]

        def compute(t, b):
            res = bias_v
            for half in range(2):
                accs = [None] * (SPC // 2)
                for j in range(NSLOT):
                    wj = [wv[pl.ds(j * D + q * 16, 16)] for q in range(NQ)]
                    for ii in range(SPC // 2):
                        i = half * (SPC // 2) + ii
                        t8 = tree_sum([
                            dbuf[b, NSLOT * i + j, pl.ds(q * 16, 16)] * wj[q]
                            for q in range(NQ)
                        ])
                        accs[ii] = t8 if accs[ii] is None else accs[ii] + t8
                for ii in range(SPC // 2):
                    i = half * (SPC // 2) + ii
                    s = accs[ii]
                    for sh in (1, 2, 4, 8):
                        s = s + s[jnp.bitwise_xor(io, sh)]
                    res = jnp.where(io == i, res + s, res)
            acc[pl.ds(t * 16, 16)] = res

        fire(0, 0)
        fire(1, 1)

        def body(it, carry):
            for b in range(2):
                t = it * 2 + b
                wait(b)
                compute(t, b)

                @pl.when(t + 2 < NCHUNK)
                def _():
                    fire(t + 2, b)

            return carry

        lax.fori_loop(0, NCHUNK // 2, body, 0)
        pltpu.sync_copy(acc, out_hbm.at[pl.ds(base, BPW)])

    return sc_k(g_feats, samples_flat, w_flat, bias16)


def kernel(g_feats, samples, W, b):
    samples_flat = samples.reshape(-1)
    w_flat = W.reshape(-1)
    bias16 = jnp.full((16,), b[0], jnp.float32)
    out_flat = _sc_kernel(g_feats, samples_flat, w_flat, bias16)
    return out_flat.reshape(B, 1)


# speedup vs baseline: 1.1160x; 1.0184x over previous
import functools

import jax
import jax.numpy as jnp
from jax import lax
from jax.experimental import pallas as pl
from jax.experimental.pallas import tpu as pltpu
from jax.experimental.pallas import tpu_sc as plsc

VOCAB = 100000
D = 128
B = 16384
NSLOT = 5
PCOL = 8

NC = 2
NS = 16
NW = NC * NS
BPW = B // NW
SUB = BPW // 128


def _tc_project_body(g_ref, w_ref, b_ref, p_ref):
    p_ref[...] = jnp.dot(
        g_ref[...].astype(jnp.bfloat16),
        w_ref[...].astype(jnp.bfloat16),
        preferred_element_type=jnp.float32,
    ) + b_ref[...]


def _tc_project(g_feats, w_pad, b_pad):
    rows = 20000
    grid = VOCAB // rows
    return pl.pallas_call(
        _tc_project_body,
        grid=(grid,),
        in_specs=[
            pl.BlockSpec((rows, D), lambda i: (i, 0)),
            pl.BlockSpec((D, PCOL), lambda i: (0, 0)),
            pl.BlockSpec((1, PCOL), lambda i: (0, 0)),
        ],
        out_specs=pl.BlockSpec((rows, PCOL), lambda i: (i, 0)),
        out_shape=jax.ShapeDtypeStruct((VOCAB, PCOL), jnp.float32),
    )(g_feats, w_pad, b_pad)


def _sc_gather(p_flat, samples_t):
    mesh = plsc.VectorSubcoreMesh(core_axis_name="c", subcore_axis_name="s")

    @functools.partial(
        pl.kernel,
        mesh=mesh,
        out_type=jax.ShapeDtypeStruct((B,), jnp.float32),
        scratch_types=[
            pltpu.VMEM((NSLOT, BPW), jnp.int32),
            pltpu.VMEM((NSLOT * SUB, 128), jnp.int32),
            pltpu.VMEM((NSLOT * SUB, 128), jnp.float32),
            pltpu.VMEM((BPW,), jnp.float32),
            pltpu.SemaphoreType.DMA,
        ],
    )
    def sc_k(pflat_hbm, st_hbm, out_hbm, sv, fidx, gbuf, acc, sem):
        wid = lax.axis_index("s") * NC + lax.axis_index("c")
        base = wid * BPW
        pltpu.sync_copy(st_hbm.at[:, pl.ds(base, BPW)], sv)
        handles = []
        for j in range(NSLOT):
            for s in range(SUB):
                row = j * SUB + s
                for c in range(8):
                    ids = sv[j, pl.ds(s * 128 + c * 16, 16)]
                    fidx[row, pl.ds(c * 16, 16)] = ids * PCOL + j
                handles.append(
                    pltpu.async_copy(pflat_hbm.at[fidx.at[row]], gbuf.at[row], sem)
                )
        for h in handles:
            h.wait()
        for c in range(BPW // 16):
            s = c // 8
            off = (c % 8) * 16
            tot = gbuf[s, pl.ds(off, 16)]
            for j in range(1, NSLOT):
                tot = tot + gbuf[j * SUB + s, pl.ds(off, 16)]
            acc[pl.ds(c * 16, 16)] = tot
        pltpu.sync_copy(acc, out_hbm.at[pl.ds(base, BPW)])

    return sc_k(p_flat, samples_t)


def kernel(g_feats, samples, W, b):
    w_pad = jnp.zeros((D, PCOL), jnp.float32).at[:, :NSLOT].set(
        W.reshape(NSLOT, D).T
    )
    b_pad = jnp.zeros((1, PCOL), jnp.float32).at[0, 0].set(b[0])
    p = _tc_project(g_feats, w_pad, b_pad)
    p_flat = p.reshape(-1)
    samples_t = samples.T
    out_flat = _sc_gather(p_flat, samples_t)
    return out_flat.reshape(B, 1)
